# Initial kernel scaffold; baseline (speedup 1.0000x reference)
import jax
import jax.numpy as jnp
from jax.experimental import pallas as pl


def kernel(point_xyz_norm, point_feats, tactile_xyz_norm, tactile_feats, triplane_feats_at_points, ctx_emb, pW1, pb1, pW2, pb2, pW3, pb3, tW1, tb1, tW2, tb2, tW3, tb3, global_token):
    raise NotImplementedError("write your pallas kernel here")


# speedup vs baseline: 15.1235x; 15.1235x over previous
import jax
import jax.numpy as jnp
import numpy as np
from jax.experimental import pallas as pl
from jax.experimental.pallas import tpu as pltpu

D_MODEL = 512
PE_BANDS = 6
PE_MAX_FREQ = 10.0
K_TACTILE = 8
TAC_TEMP = 0.05
POINT_FEAT = 64
TAC_FEAT = 32
PLANE_CH = 32
PE_DIM = 3 * 2 * PE_BANDS

BN = 512
BM = 512


def _gelu(x):
    return x * (0.5 * (1.0 + jax.lax.erf(x * (1.0 / np.sqrt(2.0)))))


def _pe_from_rep(ang):
    col = jax.lax.broadcasted_iota(jnp.int32, ang.shape, 1)
    return jnp.where((col % 12) < PE_BANDS, jnp.sin(ang), jnp.cos(ang))


def _tactile_kernel(feats_ref, xyzrep_ref, fvec_ref, ctx_ref,
                    w1_ref, b1_ref, w2_ref, b2_ref, w3_ref, b3_ref,
                    tok_ref, out_ref):
    pe = _pe_from_rep(xyzrep_ref[:] * fvec_ref[:])
    x = jnp.concatenate([feats_ref[:], pe], axis=1)
    h = _gelu(jnp.dot(x, w1_ref[:], preferred_element_type=jnp.float32) + b1_ref[:])
    h = _gelu(jnp.dot(h, w2_ref[:], preferred_element_type=jnp.float32) + b2_ref[:])
    tok = jnp.dot(h, w3_ref[:], preferred_element_type=jnp.float32) + b3_ref[:]
    tok_ref[:] = tok
    out_ref[:] = tok + ctx_ref[:]


def _point_kernel(pftri_ref, xyzrep_ref, xyz_ref, txyzT_ref, ttok_ref,
                  fvec_ref, ctx_ref,
                  w1_ref, b1_ref, w2_ref, b2_ref, w3_ref, b3_ref,
                  out_ref):
    pe = _pe_from_rep(xyzrep_ref[:] * fvec_ref[:])
    x = jnp.concatenate([pftri_ref[:], pe], axis=1)
    h = _gelu(jnp.dot(x, w1_ref[:], preferred_element_type=jnp.float32) + b1_ref[:])
    h = _gelu(jnp.dot(h, w2_ref[:], preferred_element_type=jnp.float32) + b2_ref[:])
    ptok = jnp.dot(h, w3_ref[:], preferred_element_type=jnp.float32) + b3_ref[:]

    xyz = xyz_ref[:]
    px, py, pz = xyz[:, 0:1], xyz[:, 1:2], xyz[:, 2:3]
    t = txyzT_ref[:]
    tx, ty, tz = t[0:1, :], t[1:2, :], t[2:3, :]
    p2 = px * px + py * py + pz * pz
    t2 = tx * tx + ty * ty + tz * tz
    d2 = (p2 + t2) - 2.0 * (px * tx + py * ty + pz * tz)

    work = d2
    m = None
    d2min = None
    for i in range(K_TACTILE):
        m = jnp.min(work, axis=1, keepdims=True)
        if i == 0:
            d2min = m
        if i < K_TACTILE - 1:
            work = jnp.where(work <= m, jnp.inf, work)
    thr = m

    dist = jnp.sqrt(jnp.maximum(d2, 0.0))
    dmin = jnp.sqrt(jnp.maximum(d2min, 0.0))
    logits = (dmin - dist) * (1.0 / TAC_TEMP)
    wu = jnp.where(d2 <= thr, jnp.exp(logits), 0.0)
    w = wu / jnp.sum(wu, axis=1, keepdims=True)
    treg = jnp.dot(w, ttok_ref[:], preferred_element_type=jnp.float32)
    out_ref[:] = ptok + treg + ctx_ref[:]


def kernel(point_xyz_norm, point_feats, tactile_xyz_norm, tactile_feats,
           triplane_feats_at_points, ctx_emb,
           pW1, pb1, pW2, pb2, pW3, pb3,
           tW1, tb1, tW2, tb2, tW3, tb3,
           global_token):
    B, N, _ = point_xyz_norm.shape
    M = tactile_xyz_norm.shape[1]
    f32 = jnp.float32

    freqs = jnp.linspace(1.0, PE_MAX_FREQ, PE_BANDS)
    fvec = (jnp.tile(jnp.concatenate([freqs, freqs]), 3) * np.pi).reshape(1, PE_DIM)

    p_xyzrep = jnp.repeat(point_xyz_norm, 2 * PE_BANDS, axis=-1)
    t_xyzrep = jnp.repeat(tactile_xyz_norm, 2 * PE_BANDS, axis=-1)
    pftri = jnp.concatenate([point_feats, triplane_feats_at_points], axis=-1)
    pW1r = jnp.concatenate([pW1[:POINT_FEAT], pW1[POINT_FEAT + PE_DIM:], pW1[POINT_FEAT:POINT_FEAT + PE_DIM]], axis=0)
    txyzT = jnp.transpose(tactile_xyz_norm, (0, 2, 1))
    ctx = ctx_emb[:, None, :]

    def cw(spec_shape):
        return pl.BlockSpec(spec_shape, lambda b, i: (0,) * len(spec_shape))

    in_tac = TAC_FEAT + PE_DIM
    tac_grid = (B, M // BM)
    ttok, tac_out = pl.pallas_call(
        _tactile_kernel,
        grid=tac_grid,
        in_specs=[
            pl.BlockSpec((None, BM, TAC_FEAT), lambda b, i: (b, i, 0)),
            pl.BlockSpec((None, BM, PE_DIM), lambda b, i: (b, i, 0)),
            cw((1, PE_DIM)),
            pl.BlockSpec((None, 1, D_MODEL), lambda b, i: (b, 0, 0)),
            cw((in_tac, D_MODEL)), cw((1, D_MODEL)),
            cw((D_MODEL, D_MODEL)), cw((1, D_MODEL)),
            cw((D_MODEL, D_MODEL)), cw((1, D_MODEL)),
        ],
        out_specs=[
            pl.BlockSpec((None, BM, D_MODEL), lambda b, i: (b, i, 0)),
            pl.BlockSpec((None, BM, D_MODEL), lambda b, i: (b, i, 0)),
        ],
        out_shape=[
            jax.ShapeDtypeStruct((B, M, D_MODEL), f32),
            jax.ShapeDtypeStruct((B, M, D_MODEL), f32),
        ],
    )(tactile_feats, t_xyzrep, fvec, ctx,
      tW1, tb1.reshape(1, -1), tW2, tb2.reshape(1, -1), tW3, tb3.reshape(1, -1))

    in_point = POINT_FEAT + 3 * PLANE_CH + PE_DIM
    pt_grid = (B, N // BN)
    point_out = pl.pallas_call(
        _point_kernel,
        grid=pt_grid,
        in_specs=[
            pl.BlockSpec((None, BN, POINT_FEAT + 3 * PLANE_CH), lambda b, i: (b, i, 0)),
            pl.BlockSpec((None, BN, PE_DIM), lambda b, i: (b, i, 0)),
            pl.BlockSpec((None, BN, 3), lambda b, i: (b, i, 0)),
            pl.BlockSpec((None, 3, M), lambda b, i: (b, 0, 0)),
            pl.BlockSpec((None, M, D_MODEL), lambda b, i: (b, 0, 0)),
            cw((1, PE_DIM)),
            pl.BlockSpec((None, 1, D_MODEL), lambda b, i: (b, 0, 0)),
            cw((in_point, D_MODEL)), cw((1, D_MODEL)),
            cw((D_MODEL, D_MODEL)), cw((1, D_MODEL)),
            cw((D_MODEL, D_MODEL)), cw((1, D_MODEL)),
        ],
        out_specs=pl.BlockSpec((None, BN, D_MODEL), lambda b, i: (b, i, 0)),
        out_shape=jax.ShapeDtypeStruct((B, N, D_MODEL), f32),
    )(pftri, p_xyzrep, point_xyz_norm, txyzT, ttok, fvec, ctx,
      pW1r, pb1.reshape(1, -1), pW2, pb2.reshape(1, -1), pW3, pb3.reshape(1, -1))

    global_tok = jnp.broadcast_to(global_token, (B, 1, D_MODEL)) + ctx
    return jnp.concatenate([global_tok, point_out, tac_out], axis=1)
